# baseline (speedup 1.0000x reference)
import functools
import jax
import jax.numpy as jnp
from jax import lax
from jax.experimental import pallas as pl
from jax.experimental.pallas import tpu as pltpu
from jax.experimental.pallas import tpu_sc as plsc

_BINS = 10
_ALPHA = 0.75
_N = 16777216
_NW = 32
_PER_W = _N // _NW
_CHUNK = 16384
_NCHUNK = _PER_W // _CHUNK
_L = 16
_VPC = _CHUNK // _L
_UNROLL = 8

_LOG1P_C = (
    5.62933e-07, 0.99995744, -0.4992064, 0.32697237,
    -0.22283472, 0.13076335, -0.052623954, 0.010118902,
)


def _make_sc_call():
    mesh = plsc.VectorSubcoreMesh(core_axis_name="c", subcore_axis_name="s")

    @functools.partial(
        pl.kernel,
        mesh=mesh,
        compiler_params=pltpu.CompilerParams(needs_layout_passes=False),
        out_type=jax.ShapeDtypeStruct((_NW, 32 * _L), jnp.float32),
        scratch_types=[
            pltpu.VMEM((_CHUNK,), jnp.float32),
            pltpu.VMEM((_CHUNK,), jnp.float32),
            pltpu.VMEM((32 * _L,), jnp.float32),
        ],
    )
    def _sc_hist(x_hbm, t_hbm, out_hbm, xbuf, tbuf, tab):
        c = lax.axis_index("c")
        s = lax.axis_index("s")
        wid = s * 2 + c
        base = wid * _PER_W

        zero = jnp.zeros((_L,), jnp.float32)
        for r in range(32):
            tab[pl.ds(r * _L, _L)] = zero

        lane = lax.iota(jnp.int32, _L)
        ones = jnp.full((_L,), 1.0, jnp.float32)

        def do_vec(xr, tr, j):
            off = j * _L
            xv = xr[pl.ds(off, _L)]
            tv = tr[pl.ds(off, _L)]
            ax = jnp.abs(xv)
            e = jnp.exp(-ax)
            p = jnp.full((_L,), _LOG1P_C[7], jnp.float32)
            for k in range(6, -1, -1):
                p = p * e + _LOG1P_C[k]
            loss = jnp.maximum(xv, 0.0) - xv * tv + p
            inv = 1.0 / (1.0 + e)
            pred = jnp.where(xv >= 0.0, inv, e * inv)
            g = jnp.abs(pred - tv)
            u = g * jnp.float32(_BINS)
            idx = jnp.minimum(u.astype(jnp.int32), _BINS - 1)
            fi = idx * _L + lane
            plsc.addupdate_scatter(tab, [fi], loss)
            plsc.addupdate_scatter(tab, [fi + 16 * _L], ones)

        def chunk_body(ci, carry):
            off = base + ci * _CHUNK
            pltpu.sync_copy(x_hbm.at[pl.ds(off, _CHUNK)], xbuf)
            pltpu.sync_copy(t_hbm.at[pl.ds(off, _CHUNK)], tbuf)

            def group(gi, carry2):
                for v in range(_UNROLL):
                    do_vec(xbuf, tbuf, gi * _UNROLL + v)
                return carry2

            lax.fori_loop(0, _VPC // _UNROLL, group, 0)
            return carry

        lax.fori_loop(0, _NCHUNK, chunk_body, 0)
        pltpu.sync_copy(tab, out_hbm.at[wid])

    return _sc_hist


_sc_call = _make_sc_call()


def kernel(x, target):
    parts = _sc_call(x, target).reshape(_NW, 32, _L)
    ls = jnp.sum(parts[:, 0:_BINS, :], axis=(0, 2))
    cnt = jnp.sum(parts[:, 16:16 + _BINS, :], axis=(0, 2))
    tot = jnp.clip(cnt, 1.0, None)
    w = tot ** (-_ALPHA)
    return jnp.sum(ls * w) / _N


# speedup vs baseline: 4.6736x; 4.6736x over previous
import functools
import jax
import jax.numpy as jnp
from jax import lax
from jax.experimental import pallas as pl
from jax.experimental.pallas import tpu as pltpu
from jax.experimental.pallas import tpu_sc as plsc

_BINS = 10
_ALPHA = 0.75
_N = 16777216
_NW = 32
_PER_W = _N // _NW
_CHUNK = 16384
_NCHUNK = _PER_W // _CHUNK
_L = 16
_VPC = _CHUNK // _L
_UNROLL = 8

_LOG1P_C = (
    2.2132785e-05, 0.9990102, -0.48915577, 0.2833024,
    -0.13011792, 0.030102247,
)
_DEG = len(_LOG1P_C) - 1


def _make_sc_call():
    mesh = plsc.VectorSubcoreMesh(core_axis_name="c", subcore_axis_name="s")

    @functools.partial(
        pl.kernel,
        mesh=mesh,
        compiler_params=pltpu.CompilerParams(needs_layout_passes=False),
        out_type=jax.ShapeDtypeStruct((_NW, 32 * _L), jnp.float32),
        scratch_types=[
            pltpu.VMEM((_CHUNK,), jnp.float32),
            pltpu.VMEM((_CHUNK,), jnp.float32),
            pltpu.VMEM((_CHUNK,), jnp.float32),
            pltpu.VMEM((_CHUNK,), jnp.float32),
            pltpu.VMEM((32 * _L,), jnp.float32),
            pltpu.SemaphoreType.DMA,
            pltpu.SemaphoreType.DMA,
            pltpu.SemaphoreType.DMA,
            pltpu.SemaphoreType.DMA,
        ],
    )
    def _sc_hist(x_hbm, t_hbm, out_hbm, xb0, xb1, tb0, tb1, tab,
                 sx0, sx1, st0, st1):
        c = lax.axis_index("c")
        s = lax.axis_index("s")
        wid = s * 2 + c
        base = wid * _PER_W

        zero = jnp.zeros((_L,), jnp.float32)
        for r in range(32):
            tab[pl.ds(r * _L, _L)] = zero

        lane = lax.iota(jnp.int32, _L)
        ones = jnp.full((_L,), 1.0, jnp.float32)

        def start(ci, xb, tb, sx, st):
            off = base + ci * _CHUNK
            pltpu.make_async_copy(x_hbm.at[pl.ds(off, _CHUNK)], xb, sx).start()
            pltpu.make_async_copy(t_hbm.at[pl.ds(off, _CHUNK)], tb, st).start()

        def wait(xb, tb, sx, st):
            pltpu.make_async_copy(x_hbm.at[pl.ds(0, _CHUNK)], xb, sx).wait()
            pltpu.make_async_copy(t_hbm.at[pl.ds(0, _CHUNK)], tb, st).wait()

        def compute(xr, tr):
            @plsc.parallel_loop(0, _VPC, 1, unroll=_UNROLL)
            def _vec(j):
                off = j * _L
                xv = xr[pl.ds(off, _L)]
                tv = tr[pl.ds(off, _L)]
                ax = jnp.abs(xv)
                e = jnp.exp(-ax)
                p = jnp.full((_L,), _LOG1P_C[_DEG], jnp.float32)
                for k in range(_DEG - 1, -1, -1):
                    p = p * e + _LOG1P_C[k]
                loss = jnp.maximum(xv, 0.0) - xv * tv + p
                inv = 1.0 / (1.0 + e)
                tt = jnp.where(xv >= 0.0, tv, 1.0 - tv)
                g = jnp.abs(inv - tt)
                u = g * jnp.float32(_BINS)
                idx = jnp.minimum(u.astype(jnp.int32), _BINS - 1)
                fi = idx * _L + lane
                plsc.addupdate_scatter(tab, [fi], loss)
                plsc.addupdate_scatter(tab, [fi + 16 * _L], ones)

        start(0, xb0, tb0, sx0, st0)

        def outer(k, carry):
            start(2 * k + 1, xb1, tb1, sx1, st1)
            wait(xb0, tb0, sx0, st0)
            compute(xb0, tb0)

            @pl.when(k < _NCHUNK // 2 - 1)
            def _pre():
                start(2 * k + 2, xb0, tb0, sx0, st0)

            wait(xb1, tb1, sx1, st1)
            compute(xb1, tb1)
            return carry

        lax.fori_loop(0, _NCHUNK // 2, outer, 0)
        pltpu.sync_copy(tab, out_hbm.at[wid])

    return _sc_hist


_sc_call = _make_sc_call()


def kernel(x, target):
    parts = _sc_call(x, target).reshape(_NW, 32, _L)
    ls = jnp.sum(parts[:, 0:_BINS, :], axis=(0, 2))
    cnt = jnp.sum(parts[:, 16:16 + _BINS, :], axis=(0, 2))
    tot = jnp.clip(cnt, 1.0, None)
    w = tot ** (-_ALPHA)
    return jnp.sum(ls * w) / _N
